# Initial kernel scaffold; baseline (speedup 1.0000x reference)
import jax
import jax.numpy as jnp
from jax.experimental import pallas as pl


def kernel(x, cur_time, time_step, conv_params, gat1, gat2, lin_W, lin_b):
    raise NotImplementedError("write your pallas kernel here")


# speedup vs baseline: 28.0518x; 28.0518x over previous
import jax
import jax.numpy as jnp
import numpy as np
from jax.experimental import pallas as pl
from jax.experimental.pallas import tpu as pltpu

GRID = 32
STEP = 1.0 / GRID
NUM_STEPS = 4
HEADS = 4
HID = 64
B = 64
N = B * GRID
F1 = HEADS * HID

_HI = jax.lax.Precision.HIGHEST


def _roll(h, s):
    if s == 0:
        return h
    return jnp.concatenate([h[:, -s:, :], h[:, :-s, :]], axis=1)


def _dot(a, b):
    return jnp.dot(a, b, preferred_element_type=jnp.float32)


def _dotx(a, b):
    return jnp.dot(a, b, preferred_element_type=jnp.float32, precision=_HI)


def _b32(x):
    return x.astype(jnp.bfloat16).astype(jnp.float32)


def _gather_chunks(M, X):
    outs = []
    for c in range(8):
        r = slice(256 * c, 256 * (c + 1))
        outs.append(_dotx(M[r], X[r]))
    return jnp.concatenate(outs, axis=0)


def _elu(x):
    return jnp.where(x > 0, x, jnp.exp(jnp.minimum(x, 0.0)) - 1.0)


def _leaky(x):
    return jnp.where(x >= 0, x, 0.2 * x)


def _body(ts_ref, xg_ref, xl_ref,
          w5a_ref, b5a_ref, w5b_ref, b5b_ref, w5c_ref, b5c_ref,
          w5d_ref, b5d_ref, w5e_ref, b5e_ref, w5f_ref, b5f_ref,
          wl1_ref, bl1_ref, wr1_ref, br1_ref, am1_ref, bs1_ref,
          wl2_ref, bl2_ref, wr2_ref, br2_ref, am2_ref, bs2_ref,
          hmt_ref, w2_ref, out_ref):
    ts = ts_ref[...]
    xi = -(ts + 2 * ts + 2 * ts + ts) / 6.0

    gio = jax.lax.broadcasted_iota(jnp.int32, (B, GRID, 1), 1).astype(jnp.float32)
    x_n = (gio + 0.5) * STEP + xi
    x_n = x_n - jnp.floor(x_n)
    kio = jax.lax.broadcasted_iota(jnp.int32, (B, GRID, GRID), 2).astype(jnp.float32)
    cmp = ((kio + 0.5) * STEP <= x_n).astype(jnp.int32)
    indx = jnp.sum(cmp, axis=2, keepdims=True) - 1
    src1 = indx & 31
    src2 = (indx + 1) & 31

    bio = jax.lax.broadcasted_iota(jnp.int32, (B, GRID, 1), 0)
    off = (bio & 7) * GRID
    jio = jax.lax.broadcasted_iota(jnp.int32, (B, GRID, 256), 2)
    M1 = ((src1 + off) == jio).astype(jnp.float32).reshape(N, 256)
    M2 = ((src2 + off) == jio).astype(jnp.float32).reshape(N, 256)

    sio = jax.lax.broadcasted_iota(jnp.int32, (B, GRID, GRID), 2)
    oh1 = (src1 == sio).astype(jnp.float32)
    oh2 = (src2 == sio).astype(jnp.float32)
    ohsum = oh1 + oh2
    deg = jnp.sum(ohsum, axis=1, keepdims=True)
    gio_i = jax.lax.broadcasted_iota(jnp.int32, (B, GRID, GRID), 1)
    ident = (gio_i == sio).astype(jnp.float32)

    conv = [(w5a_ref[...], b5a_ref[...]), (w5b_ref[...], b5b_ref[...]),
            (w5c_ref[...], b5c_ref[...]), (w5d_ref[...], b5d_ref[...]),
            (w5e_ref[...], b5e_ref[...]), (w5f_ref[...], b5f_ref[...])]
    gats = [(wl1_ref[...], bl1_ref[...], wr1_ref[...], br1_ref[...],
             am1_ref[...], bs1_ref[...]),
            (wl2_ref[...], bl2_ref[...], wr2_ref[...], br2_ref[...],
             am2_ref[...], bs2_ref[...])]
    hmt = hmt_ref[...]
    w2b = _b32(w2_ref[...])

    xi_ch = jnp.broadcast_to(xi * np.float32(GRID), (B, GRID, 1))

    def gat(xc2, Wl, bl, Wr, br, am, bs):
        xl = _dot(xc2, Wl) + bl
        xr = _dot(xc2, Wr) + br
        A1 = _gather_chunks(M1, xl)
        A2 = _gather_chunks(M2, xl)
        l1 = _dotx(_leaky(A1 + xr), am)
        l2 = _dotx(_leaky(A2 + xr), am)
        ls = _dotx(_leaky(xl + xr), am)
        m = jnp.maximum(jnp.maximum(l1, l2), ls)
        a1 = jnp.exp(l1 - m)
        a2 = jnp.exp(l2 - m)
        as_ = jnp.exp(ls - m)
        den = a1 + a2 + as_
        wb = lambda a: _dotx(a / den, hmt)
        return wb(a1) * A1 + wb(a2) * A2 + wb(as_) * xl + bs

    def step_fn(step, carry):
        xg3, xlane = carry
        h = jnp.concatenate([xg3, xi_ch], axis=2)
        for (W5, b5) in conv:
            X5 = jnp.concatenate([_roll(h, s) for s in (2, 1, 0, -1, -2)],
                                 axis=2)
            ci5, co = W5.shape
            hh = _dot(X5.reshape(N, ci5), W5) + b5
            h = _elu(hh).reshape(B, GRID, co)

        xc2 = h.reshape(N, 128)
        xc2 = gat(xc2, *gats[0])
        xc2 = gat(xc2, *gats[1])
        xc2 = _elu(xc2)

        xc3 = xc2.reshape(B, GRID, F1)
        v3 = jnp.sum(_b32(xc3) * w2b, axis=2, keepdims=True)
        Sv = jnp.sum(ohsum * v3, axis=1, keepdims=True)
        q = (Sv - 1.0) / deg
        q1 = jnp.sum(oh1 * q, axis=2, keepdims=True)
        q2 = jnp.sum(oh2 * q, axis=2, keepdims=True)
        xs1 = jnp.sum(oh1 * xlane, axis=2, keepdims=True)
        xs2 = jnp.sum(oh2 * xlane, axis=2, keepdims=True)
        xg3 = xs1 * (v3 - q1) + xs2 * (v3 - q2)
        xlane = jnp.sum(ident * xg3, axis=1, keepdims=True)
        out_ref[step] = xg3.reshape(N, 1)
        return (xg3, xlane)

    jax.lax.fori_loop(0, NUM_STEPS, step_fn, (xg_ref[...], xl_ref[...]))


@jax.jit
def kernel(x, cur_time, time_step, conv_params, gat1, gat2, lin_W, lin_b):
    del cur_time, lin_b
    ts3 = time_step.astype(jnp.float32).reshape(B, 1, 1)
    xg3 = x.reshape(B, GRID, 1)
    xlane = x.reshape(B, 1, GRID)

    flat_conv = []
    for (W, b) in conv_params:
        co, ci, _ = W.shape
        flat_conv.append(jnp.transpose(W, (2, 1, 0)).reshape(5 * ci, co))
        flat_conv.append(b.reshape(1, co))

    def gat_args(g):
        Wl, bl, Wr, br, att, bias = g
        attf = att.reshape(F1)
        hsel = (np.arange(F1)[:, None] // HID) == np.arange(HEADS)[None, :]
        am = attf[:, None] * jnp.asarray(hsel, dtype=jnp.float32)
        return [Wl, bl.reshape(1, F1), Wr, br.reshape(1, F1), am,
                bias.reshape(1, F1)]

    hmt = jnp.asarray((np.arange(F1)[None, :] // HID)
                      == np.arange(HEADS)[:, None], dtype=jnp.float32)
    w2 = lin_W[F1:, 0].reshape(1, 1, F1)

    args = ([ts3, xg3, xlane] + flat_conv + gat_args(gat1) + gat_args(gat2)
            + [hmt, w2])

    out = pl.pallas_call(
        _body,
        out_shape=jax.ShapeDtypeStruct((NUM_STEPS, N, 1), jnp.float32),
    )(*args)
    return out.reshape(NUM_STEPS, N).T
